# baseline (speedup 1.0000x reference)
import functools

import jax
import jax.numpy as jnp
from jax import lax
from jax.experimental import pallas as pl
from jax.experimental.pallas import tpu as pltpu
from jax.experimental.pallas import tpu_sc as plsc


def _build_mask_sc(idx2d, sc2d, L, K):
    info = plsc.get_sparse_core_info()
    NC, NS, NL = info.num_cores, info.num_subcores, info.num_lanes
    NW = NC * NS
    assert L % NW == 0 and K % NL == 0
    rows_per_w = L // NW
    k_vecs = K // NL

    mesh = plsc.VectorSubcoreMesh(core_axis_name="c", subcore_axis_name="s")

    @functools.partial(
        pl.kernel,
        mesh=mesh,
        out_type=jax.ShapeDtypeStruct((L, L), jnp.float32),
        scratch_types=[
            pltpu.VMEM((rows_per_w, K), jnp.int32),
            pltpu.VMEM((rows_per_w, K), jnp.float32),
            pltpu.VMEM((L,), jnp.float32),
        ],
        compiler_params=pltpu.CompilerParams(needs_layout_passes=False),
    )
    def scatter_kernel(idx_hbm, sc_hbm, out_hbm, idx_v, sc_v, buf):
        wid = lax.axis_index("s") * NC + lax.axis_index("c")
        base = wid * rows_per_w
        pltpu.sync_copy(idx_hbm.at[pl.ds(base, rows_per_w)], idx_v)
        pltpu.sync_copy(sc_hbm.at[pl.ds(base, rows_per_w)], sc_v)
        zeros16 = jnp.zeros((NL,), jnp.float32)
        for z in range(L // NL):
            buf[pl.ds(z * NL, NL)] = zeros16
        for r in range(rows_per_w):
            ivs = []
            for j in range(k_vecs):
                iv = idx_v[r, pl.ds(j * NL, NL)]
                sv = sc_v[r, pl.ds(j * NL, NL)]
                plsc.addupdate_scatter(buf, [iv], sv)
                ivs.append(iv)
            pltpu.sync_copy(buf, out_hbm.at[base + r])
            for iv in ivs:
                plsc.store_scatter(buf, [iv], zeros16)

    return scatter_kernel(idx2d, sc2d)


def _attn_body(q_ref, k_ref, v_ref, m_ref, o_ref, *, scale):
    h = pl.program_id(1)
    qb = q_ref[0]
    kh = k_ref[h]
    s = jax.lax.dot_general(
        qb.astype(jnp.bfloat16), kh.astype(jnp.bfloat16),
        (((1,), (1,)), ((), ())),
        preferred_element_type=jnp.float32) * scale
    p = jnp.exp(s) * m_ref[...]
    denom = jnp.sum(p, axis=1, keepdims=True) + 1e-30
    o = jax.lax.dot_general(
        p.astype(jnp.bfloat16), v_ref[h].astype(jnp.bfloat16),
        (((1,), (0,)), ((), ())),
        preferred_element_type=jnp.float32)
    o_ref[0] = o / denom


def _attention_tc(q, k, v, m, BQ=256):
    H, L, D = q.shape
    nq = L // BQ
    grid = (nq, H)
    return pl.pallas_call(
        functools.partial(_attn_body, scale=D ** -0.5),
        grid=grid,
        in_specs=[
            pl.BlockSpec((1, BQ, D), lambda i, h: (h, i, 0)),
            pl.BlockSpec((H, L, D), lambda i, h: (0, 0, 0)),
            pl.BlockSpec((H, L, D), lambda i, h: (0, 0, 0)),
            pl.BlockSpec((BQ, L), lambda i, h: (i, 0)),
        ],
        out_specs=pl.BlockSpec((1, BQ, D), lambda i, h: (h, i, 0)),
        out_shape=jax.ShapeDtypeStruct((H, L, D), jnp.float32),
        compiler_params=pltpu.CompilerParams(
            dimension_semantics=("arbitrary", "arbitrary"),
        ),
    )(q, k, v, m)


def kernel(q, k, v, topk_indices, topk_scores):
    B, H, L, D = q.shape
    K = topk_indices.shape[-1]
    assert B == 1
    idx2d = topk_indices.reshape(L, K).astype(jnp.int32)
    sc2d = topk_scores.reshape(L, K).astype(jnp.float32)
    m = _build_mask_sc(idx2d, sc2d, L, K)
    out = _attention_tc(q[0], k[0], v[0], m)
    return out[None]


# speedup vs baseline: 1.0024x; 1.0024x over previous
import functools

import jax
import jax.numpy as jnp
from jax import lax
from jax.experimental import pallas as pl
from jax.experimental.pallas import tpu as pltpu
from jax.experimental.pallas import tpu_sc as plsc


def _build_mask_sc(idx2d, sc2d, L, K):
    info = plsc.get_sparse_core_info()
    NC, NS, NL = info.num_cores, info.num_subcores, info.num_lanes
    NW = NC * NS
    assert L % NW == 0 and K % NL == 0
    rows_per_w = L // NW
    k_vecs = K // NL

    mesh = plsc.VectorSubcoreMesh(core_axis_name="c", subcore_axis_name="s")

    @functools.partial(
        pl.kernel,
        mesh=mesh,
        out_type=jax.ShapeDtypeStruct((L, L), jnp.float32),
        scratch_types=[
            pltpu.VMEM((rows_per_w, K), jnp.int32),
            pltpu.VMEM((rows_per_w, K), jnp.float32),
            pltpu.VMEM((L,), jnp.float32),
        ],
        compiler_params=pltpu.CompilerParams(
            needs_layout_passes=False, use_tc_tiling_on_sc=True),
    )
    def scatter_kernel(idx_hbm, sc_hbm, out_hbm, idx_v, sc_v, buf):
        wid = lax.axis_index("s") * NC + lax.axis_index("c")
        base = wid * rows_per_w
        pltpu.sync_copy(idx_hbm.at[pl.ds(base, rows_per_w)], idx_v)
        pltpu.sync_copy(sc_hbm.at[pl.ds(base, rows_per_w)], sc_v)
        zeros16 = jnp.zeros((NL,), jnp.float32)
        for z in range(L // NL):
            buf[pl.ds(z * NL, NL)] = zeros16
        for r in range(rows_per_w):
            ivs = []
            for j in range(k_vecs):
                iv = idx_v[r, pl.ds(j * NL, NL)]
                sv = sc_v[r, pl.ds(j * NL, NL)]
                plsc.addupdate_scatter(buf, [iv], sv)
                ivs.append(iv)
            pltpu.sync_copy(buf, out_hbm.at[base + r])
            for iv in ivs:
                plsc.store_scatter(buf, [iv], zeros16)

    return scatter_kernel(idx2d, sc2d)


def _attn_body(q_ref, k_ref, v_ref, m_ref, o_ref, *, scale):
    h = pl.program_id(1)
    qb = q_ref[0]
    kh = k_ref[h]
    s = jax.lax.dot_general(
        qb.astype(jnp.bfloat16), kh.astype(jnp.bfloat16),
        (((1,), (1,)), ((), ())),
        preferred_element_type=jnp.float32) * scale
    p = jnp.exp(s) * m_ref[...]
    denom = jnp.sum(p, axis=1, keepdims=True) + 1e-30
    o = jax.lax.dot_general(
        p.astype(jnp.bfloat16), v_ref[h].astype(jnp.bfloat16),
        (((1,), (0,)), ((), ())),
        preferred_element_type=jnp.float32)
    o_ref[0] = o / denom


def _attention_tc(q, k, v, m, BQ=256):
    H, L, D = q.shape
    nq = L // BQ
    grid = (nq, H)
    return pl.pallas_call(
        functools.partial(_attn_body, scale=D ** -0.5),
        grid=grid,
        in_specs=[
            pl.BlockSpec((1, BQ, D), lambda i, h: (h, i, 0)),
            pl.BlockSpec((H, L, D), lambda i, h: (0, 0, 0)),
            pl.BlockSpec((H, L, D), lambda i, h: (0, 0, 0)),
            pl.BlockSpec((BQ, L), lambda i, h: (i, 0)),
        ],
        out_specs=pl.BlockSpec((1, BQ, D), lambda i, h: (h, i, 0)),
        out_shape=jax.ShapeDtypeStruct((H, L, D), jnp.float32),
        compiler_params=pltpu.CompilerParams(
            dimension_semantics=("arbitrary", "arbitrary"),
        ),
    )(q, k, v, m)


def kernel(q, k, v, topk_indices, topk_scores):
    B, H, L, D = q.shape
    K = topk_indices.shape[-1]
    assert B == 1
    idx2d = topk_indices.reshape(L, K).astype(jnp.int32)
    sc2d = topk_scores.reshape(L, K).astype(jnp.float32)
    m = _build_mask_sc(idx2d, sc2d, L, K)
    out = _attention_tc(q[0], k[0], v[0], m)
    return out[None]


# speedup vs baseline: 1.1162x; 1.1135x over previous
import functools

import jax
import jax.numpy as jnp
from jax import lax
from jax.experimental import pallas as pl
from jax.experimental.pallas import tpu as pltpu
from jax.experimental.pallas import tpu_sc as plsc


def _build_mask_sc(idx2d, sc2d, L, K):
    info = plsc.get_sparse_core_info()
    NC, NS, NL = info.num_cores, info.num_subcores, info.num_lanes
    NW = NC * NS
    assert L % NW == 0 and K % NL == 0
    rows_per_w = L // NW
    k_vecs = K // NL

    mesh = plsc.VectorSubcoreMesh(core_axis_name="c", subcore_axis_name="s")

    @functools.partial(
        pl.kernel,
        mesh=mesh,
        out_type=jax.ShapeDtypeStruct((L, L), jnp.float32),
        scratch_types=[
            pltpu.VMEM((rows_per_w, K), jnp.int32),
            pltpu.VMEM((rows_per_w, K), jnp.float32),
            pltpu.VMEM((L,), jnp.float32),
        ],
        compiler_params=pltpu.CompilerParams(
            needs_layout_passes=False, use_tc_tiling_on_sc=True),
    )
    def scatter_kernel(idx_hbm, sc_hbm, out_hbm, idx_v, sc_v, buf):
        wid = lax.axis_index("s") * NC + lax.axis_index("c")
        base = wid * rows_per_w
        pltpu.sync_copy(idx_hbm.at[pl.ds(base, rows_per_w)], idx_v)
        pltpu.sync_copy(sc_hbm.at[pl.ds(base, rows_per_w)], sc_v)
        zeros16 = jnp.zeros((NL,), jnp.float32)
        for z in range(L // NL):
            buf[pl.ds(z * NL, NL)] = zeros16
        for r in range(rows_per_w):
            ivs = []
            for j in range(k_vecs):
                iv = idx_v[r, pl.ds(j * NL, NL)]
                sv = sc_v[r, pl.ds(j * NL, NL)]
                plsc.addupdate_scatter(buf, [iv], sv)
                ivs.append(iv)
            pltpu.sync_copy(buf, out_hbm.at[base + r])
            for iv in ivs:
                plsc.store_scatter(buf, [iv], zeros16)

    return scatter_kernel(idx2d, sc2d)


def _attn_body(q_ref, k_ref, v_ref, m_ref, o_ref, *, D):
    h = pl.program_id(1)
    qb = q_ref[0]
    kh = k_ref[h]
    s = jax.lax.dot_general(
        qb, kh, (((1,), (1,)), ((), ())),
        preferred_element_type=jnp.float32)
    p = jnp.exp2(s) * m_ref[...]
    r = jax.lax.dot_general(
        p.astype(jnp.bfloat16), v_ref[h], (((1,), (0,)), ((), ())),
        preferred_element_type=jnp.float32)
    o_ref[0] = r[:, :D] / (r[:, D:D + 1] + 1e-30)


def _attention_tc(q, k, vaug, m, BQ=256):
    H, L, D2 = vaug.shape
    D = q.shape[-1]
    nq = L // BQ
    grid = (nq, H)
    return pl.pallas_call(
        functools.partial(_attn_body, D=D),
        grid=grid,
        in_specs=[
            pl.BlockSpec((1, BQ, D), lambda i, h: (h, i, 0)),
            pl.BlockSpec((H, L, D), lambda i, h: (0, 0, 0)),
            pl.BlockSpec((H, L, D2), lambda i, h: (0, 0, 0)),
            pl.BlockSpec((BQ, L), lambda i, h: (i, 0)),
        ],
        out_specs=pl.BlockSpec((1, BQ, D), lambda i, h: (h, i, 0)),
        out_shape=jax.ShapeDtypeStruct((H, L, D), jnp.float32),
        compiler_params=pltpu.CompilerParams(
            dimension_semantics=("arbitrary", "arbitrary"),
        ),
    )(q, k, vaug, m)


def kernel(q, k, v, topk_indices, topk_scores):
    B, H, L, D = q.shape
    K = topk_indices.shape[-1]
    assert B == 1
    idx2d = topk_indices.reshape(L, K).astype(jnp.int32)
    sc2d = topk_scores.reshape(L, K).astype(jnp.float32)
    m = _build_mask_sc(idx2d, sc2d, L, K)
    c = (D ** -0.5) * 1.4426950408889634
    qs = (q[0] * c).astype(jnp.bfloat16)
    kb = k[0].astype(jnp.bfloat16)
    vaug = jnp.concatenate(
        [v[0], jnp.ones((H, L, 1), jnp.float32),
         jnp.zeros((H, L, D - 1), jnp.float32)], axis=-1
    ).astype(jnp.bfloat16)
    out = _attention_tc(qs, kb, vaug, m)
    return out[None]


# speedup vs baseline: 1.2265x; 1.0988x over previous
import functools

import jax
import jax.numpy as jnp
from jax import lax
from jax.experimental import pallas as pl
from jax.experimental.pallas import tpu as pltpu
from jax.experimental.pallas import tpu_sc as plsc


def _build_mask_sc(idx2d, sc2d, L, K):
    info = plsc.get_sparse_core_info()
    NC, NS, NL = info.num_cores, info.num_subcores, info.num_lanes
    NW = NC * NS
    assert L % NW == 0 and K % NL == 0
    rows_per_w = L // NW
    k_vecs = K // NL

    mesh = plsc.VectorSubcoreMesh(core_axis_name="c", subcore_axis_name="s")

    @functools.partial(
        pl.kernel,
        mesh=mesh,
        out_type=jax.ShapeDtypeStruct((L, L), jnp.float32),
        scratch_types=[
            pltpu.VMEM((rows_per_w, K), jnp.int32),
            pltpu.VMEM((rows_per_w, K), jnp.float32),
            pltpu.VMEM((L,), jnp.float32),
        ],
        compiler_params=pltpu.CompilerParams(
            needs_layout_passes=False, use_tc_tiling_on_sc=True),
    )
    def scatter_kernel(idx_hbm, sc_hbm, out_hbm, idx_v, sc_v, buf):
        wid = lax.axis_index("s") * NC + lax.axis_index("c")
        base = wid * rows_per_w
        pltpu.sync_copy(idx_hbm.at[pl.ds(base, rows_per_w)], idx_v)
        pltpu.sync_copy(sc_hbm.at[pl.ds(base, rows_per_w)], sc_v)
        zeros16 = jnp.zeros((NL,), jnp.float32)
        for z in range(L // NL):
            buf[pl.ds(z * NL, NL)] = zeros16
        for r in range(rows_per_w):
            ivs = []
            for j in range(k_vecs):
                iv = idx_v[r, pl.ds(j * NL, NL)]
                sv = sc_v[r, pl.ds(j * NL, NL)]
                plsc.addupdate_scatter(buf, [iv], sv)
                ivs.append(iv)
            pltpu.sync_copy(buf, out_hbm.at[base + r])
            for iv in ivs:
                plsc.store_scatter(buf, [iv], zeros16)

    return scatter_kernel(idx2d, sc2d)


def _attn_body(q_ref, k_ref, v_ref, m_ref, o_ref, *, D):
    h = pl.program_id(1)
    qb = q_ref[0]
    kh = k_ref[h]
    s = jax.lax.dot_general(
        qb, kh, (((1,), (1,)), ((), ())),
        preferred_element_type=jnp.float32)
    p = jnp.exp2(s) * m_ref[...]
    r = jax.lax.dot_general(
        p.astype(jnp.bfloat16), v_ref[h], (((1,), (0,)), ((), ())),
        preferred_element_type=jnp.float32)
    o_ref[0] = r[:, :D] / (r[:, D:D + 1] + 1e-30)


def _attention_tc(q, k, vaug, m, BQ=512):
    H, L, D2 = vaug.shape
    D = q.shape[-1]
    nq = L // BQ
    grid = (nq, H)
    return pl.pallas_call(
        functools.partial(_attn_body, D=D),
        grid=grid,
        in_specs=[
            pl.BlockSpec((1, BQ, D), lambda i, h: (h, i, 0)),
            pl.BlockSpec((H, L, D), lambda i, h: (0, 0, 0)),
            pl.BlockSpec((H, L, D2), lambda i, h: (0, 0, 0)),
            pl.BlockSpec((BQ, L), lambda i, h: (i, 0)),
        ],
        out_specs=pl.BlockSpec((1, BQ, D), lambda i, h: (h, i, 0)),
        out_shape=jax.ShapeDtypeStruct((H, L, D), jnp.float32),
        compiler_params=pltpu.CompilerParams(
            dimension_semantics=("arbitrary", "arbitrary"),
        ),
    )(q, k, vaug, m)


def kernel(q, k, v, topk_indices, topk_scores):
    B, H, L, D = q.shape
    K = topk_indices.shape[-1]
    assert B == 1
    idx2d = topk_indices.reshape(L, K).astype(jnp.int32)
    sc2d = topk_scores.reshape(L, K).astype(jnp.float32)
    if K < 128:
        idx2d = jnp.concatenate(
            [idx2d, jnp.zeros((L, 128 - K), jnp.int32)], axis=1)
        sc2d = jnp.concatenate(
            [sc2d, jnp.zeros((L, 128 - K), jnp.float32)], axis=1)
    m = _build_mask_sc(idx2d, sc2d, L, idx2d.shape[1])
    c = (D ** -0.5) * 1.4426950408889634
    qs = (q[0] * c).astype(jnp.bfloat16)
    kb = k[0].astype(jnp.bfloat16)
    vaug = jnp.concatenate(
        [v[0], jnp.ones((H, L, 1), jnp.float32),
         jnp.zeros((H, L, D - 1), jnp.float32)], axis=-1
    ).astype(jnp.bfloat16)
    out = _attention_tc(qs, kb, vaug, m)
    return out[None]


# speedup vs baseline: 1.2357x; 1.0075x over previous
import functools

import jax
import jax.numpy as jnp
from jax import lax
from jax.experimental import pallas as pl
from jax.experimental.pallas import tpu as pltpu
from jax.experimental.pallas import tpu_sc as plsc


def _build_mask_sc(idx2d, sc2d, L, K):
    info = plsc.get_sparse_core_info()
    NC, NS, NL = info.num_cores, info.num_subcores, info.num_lanes
    NW = NC * NS
    assert L % NW == 0 and K % NL == 0
    rows_per_w = L // NW
    k_vecs = K // NL

    mesh = plsc.VectorSubcoreMesh(core_axis_name="c", subcore_axis_name="s")

    @functools.partial(
        pl.kernel,
        mesh=mesh,
        out_type=jax.ShapeDtypeStruct((L, L), jnp.float32),
        scratch_types=[
            pltpu.VMEM((rows_per_w, K), jnp.int32),
            pltpu.VMEM((rows_per_w, K), jnp.float32),
            pltpu.VMEM((L,), jnp.float32),
        ],
        compiler_params=pltpu.CompilerParams(
            needs_layout_passes=False, use_tc_tiling_on_sc=True),
    )
    def scatter_kernel(idx_hbm, sc_hbm, out_hbm, idx_v, sc_v, buf):
        wid = lax.axis_index("s") * NC + lax.axis_index("c")
        base = wid * rows_per_w
        pltpu.sync_copy(idx_hbm.at[pl.ds(base, rows_per_w)], idx_v)
        pltpu.sync_copy(sc_hbm.at[pl.ds(base, rows_per_w)], sc_v)
        zeros16 = jnp.zeros((NL,), jnp.float32)
        for z in range(L // NL):
            buf[pl.ds(z * NL, NL)] = zeros16
        for r in range(rows_per_w):
            ivs = []
            for j in range(k_vecs):
                iv = idx_v[r, pl.ds(j * NL, NL)]
                sv = sc_v[r, pl.ds(j * NL, NL)]
                plsc.addupdate_scatter(buf, [iv], sv)
                ivs.append(iv)
            pltpu.sync_copy(buf, out_hbm.at[base + r])
            for iv in ivs:
                plsc.store_scatter(buf, [iv], zeros16)

    return scatter_kernel(idx2d, sc2d)


def _attn_body(q_ref, k_ref, v_ref, m_ref, o_ref, *, D):
    h = pl.program_id(1)
    qb = q_ref[0]
    kh = k_ref[h]
    s = jax.lax.dot_general(
        qb, kh, (((1,), (1,)), ((), ())),
        preferred_element_type=jnp.float32)
    p = jnp.exp2(s) * m_ref[...]
    r = jax.lax.dot_general(
        p.astype(jnp.bfloat16), v_ref[h], (((1,), (0,)), ((), ())),
        preferred_element_type=jnp.float32)
    o_ref[0] = r[:, :D] / (r[:, D:D + 1] + 1e-30)


def _attention_tc(q, k, vaug, m, BQ=512):
    H, L, D2 = vaug.shape
    D = q.shape[-1]
    nq = L // BQ
    grid = (nq, H)
    return pl.pallas_call(
        functools.partial(_attn_body, D=D),
        grid=grid,
        in_specs=[
            pl.BlockSpec((1, BQ, D), lambda i, h: (h, i, 0)),
            pl.BlockSpec((H, L, D), lambda i, h: (0, 0, 0)),
            pl.BlockSpec((H, L, D2), lambda i, h: (0, 0, 0)),
            pl.BlockSpec((BQ, L), lambda i, h: (i, 0)),
        ],
        out_specs=pl.BlockSpec((1, BQ, D), lambda i, h: (h, i, 0)),
        out_shape=jax.ShapeDtypeStruct((H, L, D), jnp.float32),
        compiler_params=pltpu.CompilerParams(
            dimension_semantics=("arbitrary", "arbitrary"),
        ),
    )(q, k, vaug, m)


def kernel(q, k, v, topk_indices, topk_scores):
    B, H, L, D = q.shape
    K = topk_indices.shape[-1]
    assert B == 1
    idx2d = topk_indices.reshape(L, K).astype(jnp.int32)
    sc2d = topk_scores.reshape(L, K).astype(jnp.float32)
    m = _build_mask_sc(idx2d, sc2d, L, K)
    c = (D ** -0.5) * 1.4426950408889634
    qs = (q[0] * c).astype(jnp.bfloat16)
    kb = k[0].astype(jnp.bfloat16)
    vaug = jnp.concatenate(
        [v[0], jnp.ones((H, L, 1), jnp.float32),
         jnp.zeros((H, L, D - 1), jnp.float32)], axis=-1
    ).astype(jnp.bfloat16)
    out = _attention_tc(qs, kb, vaug, m)
    return out[None]
